# Initial kernel scaffold; baseline (speedup 1.0000x reference)
import jax
import jax.numpy as jnp
from jax.experimental import pallas as pl


def kernel(x, edge_index, params):
    raise NotImplementedError("write your pallas kernel here")


# speedup vs baseline: 4.4919x; 4.4919x over previous
import dataclasses
import functools
import math

import jax
import jax.numpy as jnp
from jax import lax
from jax.experimental import pallas as pl
from jax.experimental.pallas import tpu as pltpu
from jax.experimental.pallas import tpu_sc as plsc

N = 10000
E = 320000
D = 128
AVG_LOG = math.log(33.0)

NSUB = 64
NODES_PER_S = 160
NPAD = NSUB * NODES_PER_S
CHUNK = 256
EPAD = E + 2 * CHUNK
LANES = 16
NF = D // LANES
BIG = 3.0e38


def _extract(vec, mask, zero):
    return jax.lax.reduce_sum_p.bind(
        jnp.where(mask, vec, zero), axes=(0,))


def _sc_segment_kernel(b_hbm, ssrc_hbm, sdst_hbm, wb_hbm, out_hbm,
                       idx_v, rows_v, slab_v, dst_v, wb_v):
    w = lax.axis_index("s") * 2 + lax.axis_index("c")
    pltpu.sync_copy(wb_hbm, wb_v)

    zeros = jnp.zeros((LANES,), jnp.float32)
    bigs = jnp.full((LANES,), BIG, jnp.float32)
    izero = jnp.zeros((LANES,), jnp.int32)
    lane = lax.iota(jnp.int32, LANES)
    masks = [lane == k for k in range(LANES)]

    for sub in range(2):
        g = w * 2 + sub
        n0 = g * NODES_PER_S
        gq = pl.multiple_of((g // LANES) * LANES, LANES)
        gr = g % LANES
        wvec = wb_v[pl.ds(gq, LANES)]
        wvec2 = wb_v[pl.ds(gq + LANES, LANES)]
        gmask = lane == gr
        gmask2 = lane == (gr + 1)
        e_lo = _extract(wvec, gmask, izero)
        e_hi = jnp.where(
            gr == LANES - 1,
            _extract(wvec2, masks[0], izero),
            _extract(wvec, gmask2, izero))
        base0 = (e_lo // 8) * 8
        nchunks = (e_hi - base0 + (CHUNK - 1)) // CHUNK

        def flush(cur, accs):
            off = pl.multiple_of((cur - n0) * (3 * D), LANES)
            for k in range(3 * NF):
                slab_v[pl.ds(off + k * LANES, LANES)] = accs[k]

        def chunk_body(c, carry):
            base = base0 + c * CHUNK
            pltpu.sync_copy(ssrc_hbm.at[pl.ds(base, CHUNK)], idx_v)
            pltpu.sync_copy(sdst_hbm.at[pl.ds(base, CHUNK)], dst_v)
            pltpu.sync_copy(b_hbm.at[idx_v], rows_v)
            j0 = jnp.maximum(e_lo - base, 0)
            j1 = jnp.minimum(e_hi - base, CHUNK)

            def group_body(gi, gcarry):
                dvec = dst_v[pl.ds(gi * LANES, LANES)]
                cur = gcarry[0]
                accs = list(gcarry[1:])
                for k in range(LANES):
                    j = gi * LANES + k
                    valid = (j >= j0) & (j < j1)
                    d = _extract(dvec, masks[k], izero)
                    new_seg = (d != cur) & valid

                    @pl.when(new_seg)
                    def _():
                        flush(cur, accs)

                    rrow = rows_v.at[j]
                    for f in range(NF):
                        v = rrow[pl.ds(f * LANES, LANES)]
                        s_old = jnp.where(new_seg, zeros, accs[f])
                        mn_old = jnp.where(new_seg, bigs, accs[NF + f])
                        mx_old = jnp.where(new_seg, -bigs, accs[2 * NF + f])
                        accs[f] = jnp.where(valid, s_old + v, accs[f])
                        accs[NF + f] = jnp.where(
                            valid, jnp.minimum(mn_old, v), accs[NF + f])
                        accs[2 * NF + f] = jnp.where(
                            valid, jnp.maximum(mx_old, v), accs[2 * NF + f])
                    cur = jnp.where(valid, d, cur)
                return (cur,) + tuple(accs)

            return lax.fori_loop(0, CHUNK // LANES, group_body, carry)

        init = (n0 + NODES_PER_S,) + tuple(zeros for _ in range(3 * NF))
        final = lax.fori_loop(0, nchunks, chunk_body, init)
        flush(final[0], final[1:])

        pltpu.sync_copy(slab_v.at[pl.ds(0, NODES_PER_S * 3 * D)],
                        out_hbm.at[pl.ds(n0 * 3 * D, NODES_PER_S * 3 * D)])


def _sc_segment(b, ssrc_pad, sdst_pad, wb_pad):
    mesh = plsc.VectorSubcoreMesh(core_axis_name="c", subcore_axis_name="s")
    cp = pltpu.CompilerParams()
    if "needs_layout_passes" in pltpu.CompilerParams.__dataclass_fields__:
        cp = dataclasses.replace(cp, needs_layout_passes=False)
    kern = functools.partial(
        pl.kernel,
        compiler_params=cp,
        out_type=jax.ShapeDtypeStruct((NPAD * 3 * D,), jnp.float32),
        mesh=mesh,
        scratch_types=[
            pltpu.VMEM((CHUNK,), jnp.int32),
            pltpu.VMEM((CHUNK, D), jnp.float32),
            pltpu.VMEM(((NODES_PER_S + 8) * 3 * D,), jnp.float32),
            pltpu.VMEM((CHUNK,), jnp.int32),
            pltpu.VMEM((80,), jnp.int32),
        ],
    )(_sc_segment_kernel)
    return kern(b, ssrc_pad, sdst_pad, wb_pad).reshape(NPAD, 3 * D)[:N]


ROWS_BLK = 1000
_HIGH = lax.Precision.HIGHEST


def _pre_tc_kernel(h_ref, w_ref, b_ref, a_ref, bout_ref):
    h = h_ref[...]
    wd = w_ref[0:D, :]
    ws = w_ref[D:2 * D, :]
    a_ref[...] = jax.lax.dot_general(h, wd, (((1,), (0,)), ((), ())),
                                     precision=_HIGH) + b_ref[...]
    bout_ref[...] = jax.lax.dot_general(h, ws, (((1,), (0,)), ((), ())),
                                        precision=_HIGH)


def _pre_tc(h, pre_w, pre_b):
    grid = (N // ROWS_BLK,)
    return pl.pallas_call(
        _pre_tc_kernel,
        grid=grid,
        in_specs=[
            pl.BlockSpec((ROWS_BLK, D), lambda i: (i, 0)),
            pl.BlockSpec((2 * D, D), lambda i: (0, 0)),
            pl.BlockSpec((1, D), lambda i: (0, 0)),
        ],
        out_specs=[
            pl.BlockSpec((ROWS_BLK, D), lambda i: (i, 0)),
            pl.BlockSpec((ROWS_BLK, D), lambda i: (i, 0)),
        ],
        out_shape=[
            jax.ShapeDtypeStruct((N, D), jnp.float32),
            jax.ShapeDtypeStruct((N, D), jnp.float32),
        ],
    )(h, pre_w, pre_b.reshape(1, D))


def _post_tc_kernel(h_ref, a_ref, seg_ref, r0_ref, r1_ref, pw_ref, pb_ref,
                    lw_ref, lb_ref, out_ref, *, relu):
    h = h_ref[...]
    a = a_ref[...]
    seg = seg_ref[...]
    deg = (r1_ref[...] - r0_ref[...]).astype(jnp.float32)
    has = deg > 0.0
    s = jnp.where(has, deg * a + seg[:, 0:D], 0.0)
    mn = jnp.where(has, a + seg[:, D:2 * D], 0.0)
    mx = jnp.where(has, a + seg[:, 2 * D:3 * D], 0.0)
    deg_c = jnp.maximum(deg, 1.0)
    mean = s / deg_c
    agg = jnp.concatenate([s, mean, mn, mx], axis=-1)
    log_deg = jnp.log(deg_c + 1.0)
    amp = agg * (log_deg / AVG_LOG)
    att = agg * (AVG_LOG / log_deg)
    cat = jnp.concatenate([h, agg, amp, att], axis=-1)
    t = jax.lax.dot_general(cat, pw_ref[...], (((1,), (0,)), ((), ())),
                            precision=_HIGH) + pb_ref[...]
    o = jax.lax.dot_general(t, lw_ref[...], (((1,), (0,)), ((), ())),
                            precision=_HIGH) + lb_ref[...]
    if relu:
        o = jnp.maximum(o, 0.0)
    out_ref[...] = o


def _post_tc(h, a, seg, r0, r1, post_w, post_b, lin_w, lin_b, relu):
    grid = (N // ROWS_BLK,)
    kern = functools.partial(_post_tc_kernel, relu=relu)
    return pl.pallas_call(
        kern,
        grid=grid,
        in_specs=[
            pl.BlockSpec((ROWS_BLK, D), lambda i: (i, 0)),
            pl.BlockSpec((ROWS_BLK, D), lambda i: (i, 0)),
            pl.BlockSpec((ROWS_BLK, 3 * D), lambda i: (i, 0)),
            pl.BlockSpec((ROWS_BLK, 1), lambda i: (i, 0)),
            pl.BlockSpec((ROWS_BLK, 1), lambda i: (i, 0)),
            pl.BlockSpec((13 * D, D), lambda i: (0, 0)),
            pl.BlockSpec((1, D), lambda i: (0, 0)),
            pl.BlockSpec((D, D), lambda i: (0, 0)),
            pl.BlockSpec((1, D), lambda i: (0, 0)),
        ],
        out_specs=pl.BlockSpec((ROWS_BLK, D), lambda i: (i, 0)),
        out_shape=jax.ShapeDtypeStruct((N, D), jnp.float32),
    )(h, a, seg, r0, r1, post_w, post_b.reshape(1, D), lin_w,
      lin_b.reshape(1, D))


def kernel(x, edge_index, params):
    src = edge_index[0]
    dst = edge_index[1]
    perm = jnp.argsort(dst)
    sdst = jnp.take(dst, perm)
    ssrc = jnp.take(src, perm)
    rowptr = jnp.searchsorted(sdst, jnp.arange(N + 1, dtype=jnp.int32),
                              side="left").astype(jnp.int32)
    bnds = jnp.minimum(
        jnp.arange(NSUB + 1, dtype=jnp.int32) * NODES_PER_S, N)
    wb = jnp.take(rowptr, bnds)
    wb_pad = jnp.concatenate([wb, jnp.zeros((80 - NSUB - 1,), jnp.int32)])
    pad = jnp.zeros((EPAD - E,), jnp.int32)
    ssrc_pad = jnp.concatenate([ssrc, pad])
    sdst_pad = jnp.concatenate([sdst, pad])
    r0 = rowptr[:-1].reshape(N, 1)
    r1 = rowptr[1:].reshape(N, 1)

    h = x
    for l, p in enumerate(params):
        a, b = _pre_tc(h, p["pre_W"], p["pre_b"])
        seg = _sc_segment(b, ssrc_pad, sdst_pad, wb_pad)
        h = _post_tc(h, a, seg, r0, r1, p["post_W"], p["post_b"],
                     p["lin_W"], p["lin_b"], relu=(l < len(params) - 1))
    return h


# speedup vs baseline: 4.8190x; 1.0728x over previous
import dataclasses
import functools
import math

import jax
import jax.numpy as jnp
from jax import lax
from jax.experimental import pallas as pl
from jax.experimental.pallas import tpu as pltpu
from jax.experimental.pallas import tpu_sc as plsc

N = 10000
E = 320000
D = 128
AVG_LOG = math.log(33.0)

NSUB = 64
NODES_PER_S = 160
NPAD = NSUB * NODES_PER_S
CHUNK = 256
EPAD = E + 8 * CHUNK
LANES = 16
NF = D // LANES
BIG = 3.0e38


def _extract(vec, mask, zero):
    return jax.lax.reduce_sum_p.bind(
        jnp.where(mask, vec, zero), axes=(0,))


def _sc_segment_kernel(b_hbm, ssrc_hbm, sdst_hbm, wb_hbm, out_hbm,
                       idx0_v, idx1_v, rows0_v, rows1_v, dst0_v, dst1_v,
                       slab_v, wb_v, sem0, sem1):
    w = lax.axis_index("s") * 2 + lax.axis_index("c")
    pltpu.sync_copy(wb_hbm, wb_v)
    idx_b = (idx0_v, idx1_v)
    rows_b = (rows0_v, rows1_v)
    dst_b = (dst0_v, dst1_v)
    sem_b = (sem0, sem1)

    zeros = jnp.zeros((LANES,), jnp.float32)
    bigs = jnp.full((LANES,), BIG, jnp.float32)
    izero = jnp.zeros((LANES,), jnp.int32)
    lane = lax.iota(jnp.int32, LANES)
    masks = [lane == k for k in range(LANES)]

    for sub in range(2):
        g = w * 2 + sub
        n0 = g * NODES_PER_S
        gq = pl.multiple_of((g // LANES) * LANES, LANES)
        gr = g % LANES
        wvec = wb_v[pl.ds(gq, LANES)]
        wvec2 = wb_v[pl.ds(gq + LANES, LANES)]
        gmask = lane == gr
        gmask2 = lane == (gr + 1)
        e_lo = _extract(wvec, gmask, izero)
        e_hi = jnp.where(
            gr == LANES - 1,
            _extract(wvec2, masks[0], izero),
            _extract(wvec, gmask2, izero))
        base0 = (e_lo // 8) * 8
        nchunks = (e_hi - base0 + (CHUNK - 1)) // CHUNK
        npairs = jnp.maximum((nchunks + 1) // 2, 1)

        def flush(cur, accs):
            off = pl.multiple_of((cur - n0) * (3 * D), LANES)
            for k in range(3 * NF):
                slab_v[pl.ds(off + k * LANES, LANES)] = accs[k]

        def issue(c, buf):
            base = base0 + c * CHUNK
            pltpu.sync_copy(ssrc_hbm.at[pl.ds(base, CHUNK)], idx_b[buf])
            pltpu.sync_copy(sdst_hbm.at[pl.ds(base, CHUNK)], dst_b[buf])
            pltpu.async_copy(b_hbm.at[idx_b[buf]], rows_b[buf], sem_b[buf])

        def wait(buf):
            pltpu.make_async_copy(b_hbm.at[idx_b[buf]], rows_b[buf],
                                  sem_b[buf]).wait()

        def compute(c, buf, carry):
            base = base0 + c * CHUNK
            rows_v = rows_b[buf]
            dst_v = dst_b[buf]
            j0 = jnp.maximum(e_lo - base, 0)
            j1 = jnp.minimum(e_hi - base, CHUNK)

            def group_body(gi, gcarry):
                dvec = dst_v[pl.ds(gi * LANES, LANES)]
                cur = gcarry[0]
                accs = list(gcarry[1:])
                for k in range(LANES):
                    j = gi * LANES + k
                    valid = (j >= j0) & (j < j1)
                    d = _extract(dvec, masks[k], izero)
                    new_seg = (d != cur) & valid

                    @pl.when(new_seg)
                    def _():
                        flush(cur, accs)

                    rrow = rows_v.at[j]
                    for f in range(NF):
                        v = rrow[pl.ds(f * LANES, LANES)]
                        s_old = jnp.where(new_seg, zeros, accs[f])
                        mn_old = jnp.where(new_seg, bigs, accs[NF + f])
                        mx_old = jnp.where(new_seg, -bigs, accs[2 * NF + f])
                        accs[f] = jnp.where(valid, s_old + v, accs[f])
                        accs[NF + f] = jnp.where(
                            valid, jnp.minimum(mn_old, v), accs[NF + f])
                        accs[2 * NF + f] = jnp.where(
                            valid, jnp.maximum(mx_old, v), accs[2 * NF + f])
                    cur = jnp.where(valid, d, cur)
                return (cur,) + tuple(accs)

            return lax.fori_loop(0, CHUNK // LANES, group_body, carry)

        issue(0, 0)

        def pair_body(cp, carry):
            c0 = 2 * cp
            issue(c0 + 1, 1)
            wait(0)
            carry = compute(c0, 0, carry)
            issue(c0 + 2, 0)
            wait(1)
            return compute(c0 + 1, 1, carry)

        init = (n0 + NODES_PER_S,) + tuple(zeros for _ in range(3 * NF))
        final = lax.fori_loop(0, npairs, pair_body, init)
        wait(0)
        flush(final[0], final[1:])

        pltpu.sync_copy(slab_v.at[pl.ds(0, NODES_PER_S * 3 * D)],
                        out_hbm.at[pl.ds(n0 * 3 * D, NODES_PER_S * 3 * D)])


def _sc_segment(b, ssrc_pad, sdst_pad, wb_pad):
    mesh = plsc.VectorSubcoreMesh(core_axis_name="c", subcore_axis_name="s")
    cp = pltpu.CompilerParams()
    if "needs_layout_passes" in pltpu.CompilerParams.__dataclass_fields__:
        cp = dataclasses.replace(cp, needs_layout_passes=False)
    kern = functools.partial(
        pl.kernel,
        compiler_params=cp,
        out_type=jax.ShapeDtypeStruct((NPAD * 3 * D,), jnp.float32),
        mesh=mesh,
        scratch_types=[
            pltpu.VMEM((CHUNK,), jnp.int32),
            pltpu.VMEM((CHUNK,), jnp.int32),
            pltpu.VMEM((CHUNK, D), jnp.float32),
            pltpu.VMEM((CHUNK, D), jnp.float32),
            pltpu.VMEM((CHUNK,), jnp.int32),
            pltpu.VMEM((CHUNK,), jnp.int32),
            pltpu.VMEM(((NODES_PER_S + 1) * 3 * D,), jnp.float32),
            pltpu.VMEM((80,), jnp.int32),
            pltpu.SemaphoreType.DMA,
            pltpu.SemaphoreType.DMA,
        ],
    )(_sc_segment_kernel)
    return kern(b, ssrc_pad, sdst_pad, wb_pad).reshape(NPAD, 3 * D)[:N]


ROWS_BLK = 1000
_HIGH = lax.Precision.HIGHEST


def _pre_tc_kernel(h_ref, w_ref, b_ref, a_ref, bout_ref):
    h = h_ref[...]
    wd = w_ref[0:D, :]
    ws = w_ref[D:2 * D, :]
    a_ref[...] = jax.lax.dot_general(h, wd, (((1,), (0,)), ((), ())),
                                     precision=_HIGH) + b_ref[...]
    bout_ref[...] = jax.lax.dot_general(h, ws, (((1,), (0,)), ((), ())),
                                        precision=_HIGH)


def _pre_tc(h, pre_w, pre_b):
    grid = (N // ROWS_BLK,)
    return pl.pallas_call(
        _pre_tc_kernel,
        grid=grid,
        in_specs=[
            pl.BlockSpec((ROWS_BLK, D), lambda i: (i, 0)),
            pl.BlockSpec((2 * D, D), lambda i: (0, 0)),
            pl.BlockSpec((1, D), lambda i: (0, 0)),
        ],
        out_specs=[
            pl.BlockSpec((ROWS_BLK, D), lambda i: (i, 0)),
            pl.BlockSpec((ROWS_BLK, D), lambda i: (i, 0)),
        ],
        out_shape=[
            jax.ShapeDtypeStruct((N, D), jnp.float32),
            jax.ShapeDtypeStruct((N, D), jnp.float32),
        ],
    )(h, pre_w, pre_b.reshape(1, D))


def _post_tc_kernel(h_ref, a_ref, seg_ref, r0_ref, r1_ref, pw_ref, pb_ref,
                    lw_ref, lb_ref, out_ref, *, relu):
    h = h_ref[...]
    a = a_ref[...]
    seg = seg_ref[...]
    deg = (r1_ref[...] - r0_ref[...]).astype(jnp.float32)
    has = deg > 0.0
    s = jnp.where(has, deg * a + seg[:, 0:D], 0.0)
    mn = jnp.where(has, a + seg[:, D:2 * D], 0.0)
    mx = jnp.where(has, a + seg[:, 2 * D:3 * D], 0.0)
    deg_c = jnp.maximum(deg, 1.0)
    mean = s / deg_c
    agg = jnp.concatenate([s, mean, mn, mx], axis=-1)
    log_deg = jnp.log(deg_c + 1.0)
    amp = agg * (log_deg / AVG_LOG)
    att = agg * (AVG_LOG / log_deg)
    cat = jnp.concatenate([h, agg, amp, att], axis=-1)
    t = jax.lax.dot_general(cat, pw_ref[...], (((1,), (0,)), ((), ())),
                            precision=_HIGH) + pb_ref[...]
    o = jax.lax.dot_general(t, lw_ref[...], (((1,), (0,)), ((), ())),
                            precision=_HIGH) + lb_ref[...]
    if relu:
        o = jnp.maximum(o, 0.0)
    out_ref[...] = o


def _post_tc(h, a, seg, r0, r1, post_w, post_b, lin_w, lin_b, relu):
    grid = (N // ROWS_BLK,)
    kern = functools.partial(_post_tc_kernel, relu=relu)
    return pl.pallas_call(
        kern,
        grid=grid,
        in_specs=[
            pl.BlockSpec((ROWS_BLK, D), lambda i: (i, 0)),
            pl.BlockSpec((ROWS_BLK, D), lambda i: (i, 0)),
            pl.BlockSpec((ROWS_BLK, 3 * D), lambda i: (i, 0)),
            pl.BlockSpec((ROWS_BLK, 1), lambda i: (i, 0)),
            pl.BlockSpec((ROWS_BLK, 1), lambda i: (i, 0)),
            pl.BlockSpec((13 * D, D), lambda i: (0, 0)),
            pl.BlockSpec((1, D), lambda i: (0, 0)),
            pl.BlockSpec((D, D), lambda i: (0, 0)),
            pl.BlockSpec((1, D), lambda i: (0, 0)),
        ],
        out_specs=pl.BlockSpec((ROWS_BLK, D), lambda i: (i, 0)),
        out_shape=jax.ShapeDtypeStruct((N, D), jnp.float32),
    )(h, a, seg, r0, r1, post_w, post_b.reshape(1, D), lin_w,
      lin_b.reshape(1, D))


def kernel(x, edge_index, params):
    src = edge_index[0]
    dst = edge_index[1]
    perm = jnp.argsort(dst)
    sdst = jnp.take(dst, perm)
    ssrc = jnp.take(src, perm)
    rowptr = jnp.searchsorted(sdst, jnp.arange(N + 1, dtype=jnp.int32),
                              side="left").astype(jnp.int32)
    bnds = jnp.minimum(
        jnp.arange(NSUB + 1, dtype=jnp.int32) * NODES_PER_S, N)
    wb = jnp.take(rowptr, bnds)
    wb_pad = jnp.concatenate([wb, jnp.zeros((80 - NSUB - 1,), jnp.int32)])
    pad = jnp.zeros((EPAD - E,), jnp.int32)
    ssrc_pad = jnp.concatenate([ssrc, pad])
    sdst_pad = jnp.concatenate([sdst, pad])
    r0 = rowptr[:-1].reshape(N, 1)
    r1 = rowptr[1:].reshape(N, 1)

    h = x
    for l, p in enumerate(params):
        a, b = _pre_tc(h, p["pre_W"], p["pre_b"])
        seg = _sc_segment(b, ssrc_pad, sdst_pad, wb_pad)
        h = _post_tc(h, a, seg, r0, r1, p["post_W"], p["post_b"],
                     p["lin_W"], p["lin_b"], relu=(l < len(params) - 1))
    return h


# speedup vs baseline: 4.8404x; 1.0044x over previous
import dataclasses
import functools
import math

import jax
import jax.numpy as jnp
from jax import lax
from jax.experimental import pallas as pl
from jax.experimental.pallas import tpu as pltpu
from jax.experimental.pallas import tpu_sc as plsc

N = 10000
E = 320000
D = 128
AVG_LOG = math.log(33.0)

NSUB = 64
NODES_PER_S = 160
NPAD = NSUB * NODES_PER_S
CHUNK = 256
EPAD = E + 8 * CHUNK
LANES = 16
NF = D // LANES
BIG = 3.0e38


def _extract(vec, mask, zero):
    return jax.lax.reduce_sum_p.bind(
        jnp.where(mask, vec, zero), axes=(0,))


def _sc_segment_kernel(b_hbm, ssrc_hbm, sdst_hbm, wb_hbm, out_hbm,
                       idx0_v, idx1_v, rows0_v, rows1_v, dst0_v, dst1_v,
                       slab_v, wb_v, sem0, sem1):
    w = lax.axis_index("s") * 2 + lax.axis_index("c")
    pltpu.sync_copy(wb_hbm, wb_v)
    idx_b = (idx0_v, idx1_v)
    rows_b = (rows0_v, rows1_v)
    dst_b = (dst0_v, dst1_v)
    sem_b = (sem0, sem1)

    zeros = jnp.zeros((LANES,), jnp.float32)
    bigs = jnp.full((LANES,), BIG, jnp.float32)
    izero = jnp.zeros((LANES,), jnp.int32)
    lane = lax.iota(jnp.int32, LANES)
    masks = [lane == k for k in range(LANES)]

    for sub in range(2):
        g = w * 2 + sub
        n0 = g * NODES_PER_S
        gq = pl.multiple_of((g // LANES) * LANES, LANES)
        gr = g % LANES
        wvec = wb_v[pl.ds(gq, LANES)]
        wvec2 = wb_v[pl.ds(gq + LANES, LANES)]
        gmask = lane == gr
        gmask2 = lane == (gr + 1)
        e_lo = _extract(wvec, gmask, izero)
        e_hi = jnp.where(
            gr == LANES - 1,
            _extract(wvec2, masks[0], izero),
            _extract(wvec, gmask2, izero))
        base0 = (e_lo // CHUNK) * CHUNK
        nchunks = (e_hi - base0 + (CHUNK - 1)) // CHUNK
        npairs = jnp.maximum((nchunks + 1) // 2, 1)

        def flush(cur, accs):
            r = cur - n0
            owned = (r >= 0) & (r < NODES_PER_S)
            row = jnp.where(owned, r, NODES_PER_S)
            off = pl.multiple_of(row * (3 * D), LANES)
            for k in range(3 * NF):
                slab_v[pl.ds(off + k * LANES, LANES)] = accs[k]

        def issue(c, buf):
            base = base0 + c * CHUNK
            pltpu.sync_copy(ssrc_hbm.at[pl.ds(base, CHUNK)], idx_b[buf])
            pltpu.sync_copy(sdst_hbm.at[pl.ds(base, CHUNK)], dst_b[buf])
            pltpu.async_copy(b_hbm.at[idx_b[buf]], rows_b[buf], sem_b[buf])

        def wait(buf):
            pltpu.make_async_copy(b_hbm.at[idx_b[buf]], rows_b[buf],
                                  sem_b[buf]).wait()

        def compute(c, buf, carry):
            rows_v = rows_b[buf]
            dst_v = dst_b[buf]

            def group_body(gi, gcarry):
                dvec = dst_v[pl.ds(gi * LANES, LANES)]
                cur = gcarry[0]
                accs = list(gcarry[1:])
                for k in range(LANES):
                    j = gi * LANES + k
                    d = _extract(dvec, masks[k], izero)
                    new_seg = d != cur

                    @pl.when(new_seg)
                    def _():
                        flush(cur, accs)

                    rrow = rows_v.at[j]
                    for f in range(NF):
                        v = rrow[pl.ds(f * LANES, LANES)]
                        accs[f] = jnp.where(new_seg, v, accs[f] + v)
                        accs[NF + f] = jnp.where(
                            new_seg, v, jnp.minimum(accs[NF + f], v))
                        accs[2 * NF + f] = jnp.where(
                            new_seg, v, jnp.maximum(accs[2 * NF + f], v))
                    cur = d
                return (cur,) + tuple(accs)

            return lax.fori_loop(0, CHUNK // LANES, group_body, carry)

        issue(0, 0)

        def pair_body(cp, carry):
            c0 = 2 * cp
            issue(c0 + 1, 1)
            wait(0)
            carry = compute(c0, 0, carry)
            issue(c0 + 2, 0)
            wait(1)
            return compute(c0 + 1, 1, carry)

        init = (n0 + NODES_PER_S,) + tuple(zeros for _ in range(3 * NF))
        final = lax.fori_loop(0, npairs, pair_body, init)
        wait(0)
        flush(final[0], final[1:])

        pltpu.sync_copy(slab_v.at[pl.ds(0, NODES_PER_S * 3 * D)],
                        out_hbm.at[pl.ds(n0 * 3 * D, NODES_PER_S * 3 * D)])


def _sc_segment(b, ssrc_pad, sdst_pad, wb_pad):
    mesh = plsc.VectorSubcoreMesh(core_axis_name="c", subcore_axis_name="s")
    cp = pltpu.CompilerParams()
    if "needs_layout_passes" in pltpu.CompilerParams.__dataclass_fields__:
        cp = dataclasses.replace(cp, needs_layout_passes=False)
    kern = functools.partial(
        pl.kernel,
        compiler_params=cp,
        out_type=jax.ShapeDtypeStruct((NPAD * 3 * D,), jnp.float32),
        mesh=mesh,
        scratch_types=[
            pltpu.VMEM((CHUNK,), jnp.int32),
            pltpu.VMEM((CHUNK,), jnp.int32),
            pltpu.VMEM((CHUNK, D), jnp.float32),
            pltpu.VMEM((CHUNK, D), jnp.float32),
            pltpu.VMEM((CHUNK,), jnp.int32),
            pltpu.VMEM((CHUNK,), jnp.int32),
            pltpu.VMEM(((NODES_PER_S + 1) * 3 * D,), jnp.float32),
            pltpu.VMEM((80,), jnp.int32),
            pltpu.SemaphoreType.DMA,
            pltpu.SemaphoreType.DMA,
        ],
    )(_sc_segment_kernel)
    return kern(b, ssrc_pad, sdst_pad, wb_pad).reshape(NPAD, 3 * D)[:N]


ROWS_BLK = 1000
_HIGH = lax.Precision.HIGHEST


def _pre_tc_kernel(h_ref, w_ref, b_ref, a_ref, bout_ref):
    h = h_ref[...]
    wd = w_ref[0:D, :]
    ws = w_ref[D:2 * D, :]
    a_ref[...] = jax.lax.dot_general(h, wd, (((1,), (0,)), ((), ())),
                                     precision=_HIGH) + b_ref[...]
    bout_ref[...] = jax.lax.dot_general(h, ws, (((1,), (0,)), ((), ())),
                                        precision=_HIGH)


def _pre_tc(h, pre_w, pre_b):
    grid = (N // ROWS_BLK,)
    return pl.pallas_call(
        _pre_tc_kernel,
        grid=grid,
        in_specs=[
            pl.BlockSpec((ROWS_BLK, D), lambda i: (i, 0)),
            pl.BlockSpec((2 * D, D), lambda i: (0, 0)),
            pl.BlockSpec((1, D), lambda i: (0, 0)),
        ],
        out_specs=[
            pl.BlockSpec((ROWS_BLK, D), lambda i: (i, 0)),
            pl.BlockSpec((ROWS_BLK, D), lambda i: (i, 0)),
        ],
        out_shape=[
            jax.ShapeDtypeStruct((N, D), jnp.float32),
            jax.ShapeDtypeStruct((N, D), jnp.float32),
        ],
    )(h, pre_w, pre_b.reshape(1, D))


def _post_tc_kernel(h_ref, a_ref, seg_ref, r0_ref, r1_ref, pw_ref, pb_ref,
                    lw_ref, lb_ref, out_ref, *, relu):
    h = h_ref[...]
    a = a_ref[...]
    seg = seg_ref[...]
    deg = (r1_ref[...] - r0_ref[...]).astype(jnp.float32)
    has = deg > 0.0
    s = jnp.where(has, deg * a + seg[:, 0:D], 0.0)
    mn = jnp.where(has, a + seg[:, D:2 * D], 0.0)
    mx = jnp.where(has, a + seg[:, 2 * D:3 * D], 0.0)
    deg_c = jnp.maximum(deg, 1.0)
    mean = s / deg_c
    agg = jnp.concatenate([s, mean, mn, mx], axis=-1)
    log_deg = jnp.log(deg_c + 1.0)
    amp = agg * (log_deg / AVG_LOG)
    att = agg * (AVG_LOG / log_deg)
    cat = jnp.concatenate([h, agg, amp, att], axis=-1)
    t = jax.lax.dot_general(cat, pw_ref[...], (((1,), (0,)), ((), ())),
                            precision=_HIGH) + pb_ref[...]
    o = jax.lax.dot_general(t, lw_ref[...], (((1,), (0,)), ((), ())),
                            precision=_HIGH) + lb_ref[...]
    if relu:
        o = jnp.maximum(o, 0.0)
    out_ref[...] = o


def _post_tc(h, a, seg, r0, r1, post_w, post_b, lin_w, lin_b, relu):
    grid = (N // ROWS_BLK,)
    kern = functools.partial(_post_tc_kernel, relu=relu)
    return pl.pallas_call(
        kern,
        grid=grid,
        in_specs=[
            pl.BlockSpec((ROWS_BLK, D), lambda i: (i, 0)),
            pl.BlockSpec((ROWS_BLK, D), lambda i: (i, 0)),
            pl.BlockSpec((ROWS_BLK, 3 * D), lambda i: (i, 0)),
            pl.BlockSpec((ROWS_BLK, 1), lambda i: (i, 0)),
            pl.BlockSpec((ROWS_BLK, 1), lambda i: (i, 0)),
            pl.BlockSpec((13 * D, D), lambda i: (0, 0)),
            pl.BlockSpec((1, D), lambda i: (0, 0)),
            pl.BlockSpec((D, D), lambda i: (0, 0)),
            pl.BlockSpec((1, D), lambda i: (0, 0)),
        ],
        out_specs=pl.BlockSpec((ROWS_BLK, D), lambda i: (i, 0)),
        out_shape=jax.ShapeDtypeStruct((N, D), jnp.float32),
    )(h, a, seg, r0, r1, post_w, post_b.reshape(1, D), lin_w,
      lin_b.reshape(1, D))


def kernel(x, edge_index, params):
    src = edge_index[0]
    dst = edge_index[1]
    perm = jnp.argsort(dst)
    sdst = jnp.take(dst, perm)
    ssrc = jnp.take(src, perm)
    rowptr = jnp.searchsorted(sdst, jnp.arange(N + 1, dtype=jnp.int32),
                              side="left").astype(jnp.int32)
    bnds = jnp.minimum(
        jnp.arange(NSUB + 1, dtype=jnp.int32) * NODES_PER_S, N)
    wb = jnp.take(rowptr, bnds)
    wb_pad = jnp.concatenate([wb, jnp.zeros((80 - NSUB - 1,), jnp.int32)])
    ssrc_pad = jnp.concatenate([ssrc, jnp.zeros((EPAD - E,), jnp.int32)])
    sdst_pad = jnp.concatenate(
        [sdst, jnp.full((EPAD - E,), NPAD, jnp.int32)])
    r0 = rowptr[:-1].reshape(N, 1)
    r1 = rowptr[1:].reshape(N, 1)

    h = x
    for l, p in enumerate(params):
        a, b = _pre_tc(h, p["pre_W"], p["pre_b"])
        seg = _sc_segment(b, ssrc_pad, sdst_pad, wb_pad)
        h = _post_tc(h, a, seg, r0, r1, p["post_W"], p["post_b"],
                     p["lin_W"], p["lin_b"], relu=(l < len(params) - 1))
    return h
